# baseline (speedup 1.0000x reference)
import functools

import jax
import jax.numpy as jnp
from jax import lax
from jax.experimental import pallas as pl
from jax.experimental.pallas import tpu as pltpu
from jax.experimental.pallas import tpu_sc as plsc

_N = 10000
_E = 320000
_D = 128
_H = 128
_C = 40

_NC = 2
_NS = 16
_EPT = 10000
_NACC = 10016
_RPT = _NACC // _NS
_W0 = 128
_WSL = 64
_DW = 16
_DR = 640
_W1 = 48
_K = 125
_CH = 16
_NBLK0 = 2 * _EPT // _K
_NCH = _NBLK0 // _CH
_NBLK1 = _EPT // _K


def _fill(buf, rows, width, value, dtype, lanes):
  z = jnp.full((lanes,), value, dtype)
  cpr = width // lanes

  def b(i, carry):
    buf[i // cpr, pl.ds((i % cpr) * lanes, lanes)] = z
    return carry

  lax.fori_loop(0, rows * cpr, b, 0)


def _zero_slice(dst_sh, buf, k, sid):
  base = sid * _RPT
  nfull = _RPT // k
  rem = _RPT - nfull * k
  for t in range(nfull):
    pltpu.sync_copy(buf, dst_sh.at[pl.ds(base + t * k, k)])
  if rem:
    pltpu.sync_copy(buf.at[pl.ds(0, rem)],
                    dst_sh.at[pl.ds(base + nfull * k, rem)])


def _get_mesh():
  return plsc.VectorSubcoreMesh(core_axis_name="c", subcore_axis_name="s",
                                num_cores=_NC, num_subcores=_NS)


def _make_seg0():
  @functools.partial(
      pl.kernel,
      mesh=_get_mesh(),
      compiler_params=pltpu.CompilerParams(use_tc_tiling_on_sc=False),
      out_type=(
          jax.ShapeDtypeStruct((_NACC, _W0), jnp.bfloat16),
          jax.ShapeDtypeStruct((_NC, _NACC, _DW), jnp.float32),
      ),
      scratch_types=[
          pltpu.VMEM((_CH, _K), jnp.int32),
          pltpu.VMEM((_CH, _K), jnp.int32),
          pltpu.VMEM((_CH, _K), jnp.int32),
          pltpu.VMEM((_CH, _K), jnp.int32),
          pltpu.VMEM((_K, _WSL), jnp.bfloat16),
          pltpu.VMEM((_K, _WSL), jnp.bfloat16),
          pltpu.VMEM((_K, _DW), jnp.float32),
          pltpu.VMEM_SHARED((_NACC, _WSL), jnp.bfloat16),
          pltpu.VMEM_SHARED((_NACC, _WSL), jnp.bfloat16),
          pltpu.VMEM_SHARED((_NACC, _DW), jnp.float32),
          pltpu.SemaphoreType.DMA,
          pltpu.SemaphoreType.DMA,
          pltpu.SemaphoreType.DMA,
          pltpu.SemaphoreType.DMA,
      ],
  )
  def seg(table_hbm, src_hbm, dst_hbm, out_hbm, deg_hbm,
          sbuf0, dbuf0, sbuf1, dbuf1, rows0_v, rows1_v, ones_v,
          table_sh, acc_sh, deg_sh, isem0, isem1, rsem0, rsem1):
    cid = lax.axis_index("c")
    sid = lax.axis_index("s")
    col0 = cid * _WSL

    def load_idx(c, sbuf, dbuf, isem):
      pltpu.async_copy(src_hbm.at[sid, pl.ds(c * _CH, _CH)], sbuf, isem)
      pltpu.async_copy(dst_hbm.at[sid, pl.ds(c * _CH, _CH)], dbuf, isem)

    def wait_idx(c, sbuf, dbuf, isem):
      pltpu.make_async_copy(src_hbm.at[sid, pl.ds(c * _CH, _CH)], sbuf,
                            isem).wait()
      pltpu.make_async_copy(dst_hbm.at[sid, pl.ds(c * _CH, _CH)], dbuf,
                            isem).wait()

    load_idx(0, sbuf0, dbuf0, isem0)

    @pl.when(sid < _NS - 1)
    def _():
      pltpu.sync_copy(
          table_hbm.at[pl.ds(sid * _RPT, _RPT), pl.ds(col0, _WSL)],
          table_sh.at[pl.ds(sid * _RPT, _RPT)])

    @pl.when(sid == _NS - 1)
    def _():
      last = _N - (_NS - 1) * _RPT
      pltpu.sync_copy(
          table_hbm.at[pl.ds((_NS - 1) * _RPT, last), pl.ds(col0, _WSL)],
          table_sh.at[pl.ds((_NS - 1) * _RPT, last)])
    _fill(rows0_v, _K, _WSL, 0, jnp.bfloat16, 32)
    _zero_slice(acc_sh, rows0_v, _K, sid)
    _fill(ones_v, _K, _DW, 0.0, jnp.float32, 16)
    _zero_slice(deg_sh, ones_v, _K, sid)
    _fill(ones_v, _K, _DW, 1.0, jnp.float32, 16)
    plsc.subcore_barrier()

    def scat(rows_v, dref, parity):
      pltpu.sync_copy(rows_v, acc_sh.at[dref], add=True)

      @pl.when(cid == parity)
      def _():
        pltpu.sync_copy(ones_v, deg_sh.at[dref], add=True)

    def chunk(c, sbuf, dbuf):
      pltpu.async_copy(table_sh.at[sbuf.at[0]], rows0_v, rsem0)

      def pair(i, carry):
        pltpu.async_copy(table_sh.at[sbuf.at[2 * i + 1]], rows1_v, rsem1)
        pltpu.make_async_copy(table_sh.at[sbuf.at[2 * i]], rows0_v,
                              rsem0).wait()
        scat(rows0_v, dbuf.at[2 * i], 0)

        @pl.when(2 * i + 2 < _CH)
        def _():
          pltpu.async_copy(table_sh.at[sbuf.at[2 * i + 2]], rows0_v, rsem0)

        pltpu.make_async_copy(table_sh.at[sbuf.at[2 * i + 1]], rows1_v,
                              rsem1).wait()
        scat(rows1_v, dbuf.at[2 * i + 1], 1)
        return carry

      lax.fori_loop(0, _CH // 2, pair, 0)

    def outer(m, carry):
      c0 = 2 * m
      c1 = 2 * m + 1
      wait_idx(c0, sbuf0, dbuf0, isem0)
      load_idx(c1, sbuf1, dbuf1, isem1)
      chunk(c0, sbuf0, dbuf0)
      wait_idx(c1, sbuf1, dbuf1, isem1)

      @pl.when(c1 + 1 < _NCH)
      def _():
        load_idx(c1 + 1, sbuf0, dbuf0, isem0)

      chunk(c1, sbuf1, dbuf1)
      return carry

    lax.fori_loop(0, _NCH // 2, outer, 0)

    plsc.subcore_barrier()
    rsl = pl.ds(sid * _RPT, _RPT)
    pltpu.sync_copy(acc_sh.at[rsl], out_hbm.at[rsl, pl.ds(col0, _WSL)])
    pltpu.sync_copy(deg_sh.at[rsl], deg_hbm.at[cid, rsl])

  return seg


def _make_seg1():
  @functools.partial(
      pl.kernel,
      mesh=_get_mesh(),
      compiler_params=pltpu.CompilerParams(use_tc_tiling_on_sc=False),
      out_type=jax.ShapeDtypeStruct((_NC, _NACC, _W1), jnp.bfloat16),
      scratch_types=[
          pltpu.VMEM((_NBLK1, _K), jnp.int32),
          pltpu.VMEM((_NBLK1, _K), jnp.int32),
          pltpu.VMEM((_K, _W1), jnp.bfloat16),
          pltpu.VMEM((_K, _W1), jnp.bfloat16),
          pltpu.VMEM_SHARED((_NACC, _W1), jnp.bfloat16),
          pltpu.VMEM_SHARED((_NACC, _W1), jnp.bfloat16),
          pltpu.SemaphoreType.DMA,
          pltpu.SemaphoreType.DMA,
      ],
  )
  def seg(table_hbm, src_hbm, dst_hbm, out_hbm,
          src_v, dst_v, rows0_v, rows1_v, table_sh, acc_sh, sem0, sem1):
    cid = lax.axis_index("c")
    sid = lax.axis_index("s")

    pltpu.sync_copy(src_hbm.at[cid, sid], src_v)
    pltpu.sync_copy(dst_hbm.at[cid, sid], dst_v)
    pltpu.sync_copy(table_hbm.at[pl.ds(sid * _RPT, _RPT)],
                    table_sh.at[pl.ds(sid * _RPT, _RPT)])
    z32 = jnp.zeros((32,), jnp.bfloat16)

    def zrow(i, carry):
      rows0_v[i, pl.ds(0, 32)] = z32
      rows0_v[i, pl.ds(_W1 - 32, 32)] = z32
      return carry

    lax.fori_loop(0, _K, zrow, 0)
    _zero_slice(acc_sh, rows0_v, _K, sid)
    plsc.subcore_barrier()

    pltpu.async_copy(table_sh.at[src_v.at[0]], rows0_v, sem0)

    def pair(i, carry):
      j = 2 * i
      pltpu.async_copy(table_sh.at[src_v.at[j + 1]], rows1_v, sem1)
      pltpu.make_async_copy(table_sh.at[src_v.at[j]], rows0_v, sem0).wait()
      pltpu.sync_copy(rows0_v, acc_sh.at[dst_v.at[j]], add=True)

      @pl.when(j + 2 < _NBLK1)
      def _():
        pltpu.async_copy(table_sh.at[src_v.at[j + 2]], rows0_v, sem0)

      pltpu.make_async_copy(table_sh.at[src_v.at[j + 1]], rows1_v,
                            sem1).wait()
      pltpu.sync_copy(rows1_v, acc_sh.at[dst_v.at[j + 1]], add=True)
      return carry

    lax.fori_loop(0, _NBLK1 // 2, pair, 0)

    plsc.subcore_barrier()
    pltpu.sync_copy(acc_sh.at[pl.ds(sid * _RPT, _RPT)],
                    out_hbm.at[cid, pl.ds(sid * _RPT, _RPT)])

  return seg


_sc_cache = {}


def _sc(kind):
  if kind not in _sc_cache:
    _sc_cache[kind] = _make_seg0() if kind == "seg0" else _make_seg1()
  return _sc_cache[kind]


_R = 1000


def _layerB_body(x_ref, p0_ref, deg_ref, ws0_ref, wn0_ref, b0_ref, ws1_ref,
                 wn1_ref, b1_ref, t1_ref, hse_ref):
  agg = p0_ref[...].astype(jnp.float32)
  deg = deg_ref[0, :, 0:1] + deg_ref[1, :, 0:1]
  inv = 1.0 / jnp.maximum(deg, 1.0)
  hn = jnp.dot(agg * inv, wn0_ref[...], preferred_element_type=jnp.float32)
  hself = jnp.dot(x_ref[...], ws0_ref[...].astype(jnp.bfloat16),
                  preferred_element_type=jnp.float32)
  h = hself + hn + b0_ref[...][None, :]
  h = jnp.maximum(h, 0.0)
  t1 = jnp.dot(h, wn1_ref[...], preferred_element_type=jnp.float32)
  hs = jnp.dot(h, ws1_ref[...], preferred_element_type=jnp.float32)
  hs = hs + b1_ref[...][None, :]
  t1_ref[...] = jnp.concatenate(
      [t1, jnp.zeros((_R, _W1 - _C), jnp.float32)], axis=1
  ).astype(jnp.bfloat16)
  zpad = jnp.zeros((_R, 7), jnp.float32)
  hse_ref[...] = jnp.concatenate([hs, inv, zpad], axis=1).astype(jnp.bfloat16)


def _layerC_body(p1_ref, hse_ref, o_ref):
  s = p1_ref[0].astype(jnp.float32) + p1_ref[1].astype(jnp.float32)
  hse = hse_ref[...].astype(jnp.float32)
  inv = hse[:, _C:_C + 1]
  o_ref[...] = hse[:, :_C] + s[:, :_C] * inv


def kernel(features, edge_index, W_self0, W_neigh0, b0, W_self1, W_neigh1, b1):
  src = edge_index[0]
  dst = edge_index[1]
  src_cs = src.reshape(_NS, _NBLK0, _K)
  dst_cs = dst.reshape(_NS, _NBLK0, _K)
  src_r = src.reshape(_NC, _NS, _NBLK1, _K)
  dst_r = dst.reshape(_NC, _NS, _NBLK1, _K)

  nb = _N // _R

  xb = features.astype(jnp.bfloat16)
  p0, deg = _sc("seg0")(xb, src_cs, dst_cs)

  t1, hse = pl.pallas_call(
      _layerB_body,
      grid=(nb,),
      in_specs=[
          pl.BlockSpec((_R, _D), lambda i: (i, 0)),
          pl.BlockSpec((_R, _W0), lambda i: (i, 0)),
          pl.BlockSpec((_NC, _R, _DW), lambda i: (0, i, 0)),
          pl.BlockSpec((_D, _H), lambda i: (0, 0)),
          pl.BlockSpec((_D, _H), lambda i: (0, 0)),
          pl.BlockSpec((_H,), lambda i: (0,)),
          pl.BlockSpec((_H, _C), lambda i: (0, 0)),
          pl.BlockSpec((_H, _C), lambda i: (0, 0)),
          pl.BlockSpec((_C,), lambda i: (0,)),
      ],
      out_specs=[
          pl.BlockSpec((_R, _W1), lambda i: (i, 0)),
          pl.BlockSpec((_R, _C + 8), lambda i: (i, 0)),
      ],
      out_shape=[
          jax.ShapeDtypeStruct((_NACC, _W1), jnp.bfloat16),
          jax.ShapeDtypeStruct((_N, _C + 8), jnp.bfloat16),
      ],
  )(xb, p0, deg, W_self0, W_neigh0, b0, W_self1, W_neigh1, b1)

  p1 = _sc("seg1")(t1, src_r, dst_r)

  out = pl.pallas_call(
      _layerC_body,
      grid=(nb,),
      in_specs=[
          pl.BlockSpec((_NC, _R, _W1), lambda i: (0, i, 0)),
          pl.BlockSpec((_R, _C + 8), lambda i: (i, 0)),
      ],
      out_specs=pl.BlockSpec((_R, _C), lambda i: (i, 0)),
      out_shape=jax.ShapeDtypeStruct((_N, _C), jnp.float32),
  )(p1, hse)

  return out


# speedup vs baseline: 1.0260x; 1.0260x over previous
import functools

import jax
import jax.numpy as jnp
from jax import lax
from jax.experimental import pallas as pl
from jax.experimental.pallas import tpu as pltpu
from jax.experimental.pallas import tpu_sc as plsc

_N = 10000
_E = 320000
_D = 128
_H = 128
_C = 40

_NC = 2
_NS = 16
_EPT = 10240
_EPAD = _NC * _NS * _EPT
_NACC = 10016
_RPT = _NACC // _NS
_W0 = 128
_WSL = 64
_DW = 16
_DR = 640
_W1 = 48
_K = 128
_CH = 16
_NBLK0 = 2 * _EPT // _K
_NCH = _NBLK0 // _CH
_NBLK1 = _EPT // _K


def _fill(buf, rows, width, value, dtype, lanes):
  z = jnp.full((lanes,), value, dtype)
  cpr = width // lanes

  def b(i, carry):
    buf[i // cpr, pl.ds((i % cpr) * lanes, lanes)] = z
    return carry

  lax.fori_loop(0, rows * cpr, b, 0)


def _zero_slice(dst_sh, buf, k, sid):
  base = sid * _RPT
  nfull = _RPT // k
  rem = _RPT - nfull * k
  for t in range(nfull):
    pltpu.sync_copy(buf, dst_sh.at[pl.ds(base + t * k, k)])
  if rem:
    pltpu.sync_copy(buf.at[pl.ds(0, rem)],
                    dst_sh.at[pl.ds(base + nfull * k, rem)])


def _get_mesh():
  return plsc.VectorSubcoreMesh(core_axis_name="c", subcore_axis_name="s",
                                num_cores=_NC, num_subcores=_NS)


def _make_seg0():
  @functools.partial(
      pl.kernel,
      mesh=_get_mesh(),
      compiler_params=pltpu.CompilerParams(use_tc_tiling_on_sc=False),
      out_type=(
          jax.ShapeDtypeStruct((_NACC, _W0), jnp.bfloat16),
          jax.ShapeDtypeStruct((_NC, _NACC, _DW), jnp.float32),
      ),
      scratch_types=[
          pltpu.VMEM((_CH, _K), jnp.int32),
          pltpu.VMEM((_CH, _K), jnp.int32),
          pltpu.VMEM((_CH, _K), jnp.int32),
          pltpu.VMEM((_CH, _K), jnp.int32),
          pltpu.VMEM((_K, _WSL), jnp.bfloat16),
          pltpu.VMEM((_K, _WSL), jnp.bfloat16),
          pltpu.VMEM((_K, _DW), jnp.float32),
          pltpu.VMEM_SHARED((_NACC, _WSL), jnp.bfloat16),
          pltpu.VMEM_SHARED((_NACC, _WSL), jnp.bfloat16),
          pltpu.VMEM_SHARED((_NACC, _DW), jnp.float32),
          pltpu.SemaphoreType.DMA,
          pltpu.SemaphoreType.DMA,
          pltpu.SemaphoreType.DMA,
          pltpu.SemaphoreType.DMA,
      ],
  )
  def seg(table_hbm, src_hbm, dst_hbm, out_hbm, deg_hbm,
          sbuf0, dbuf0, sbuf1, dbuf1, rows0_v, rows1_v, ones_v,
          table_sh, acc_sh, deg_sh, isem0, isem1, rsem0, rsem1):
    cid = lax.axis_index("c")
    sid = lax.axis_index("s")
    col0 = cid * _WSL

    def load_idx(c, sbuf, dbuf, isem):
      pltpu.async_copy(src_hbm.at[sid, pl.ds(c * _CH, _CH)], sbuf, isem)
      pltpu.async_copy(dst_hbm.at[sid, pl.ds(c * _CH, _CH)], dbuf, isem)

    def wait_idx(c, sbuf, dbuf, isem):
      pltpu.make_async_copy(src_hbm.at[sid, pl.ds(c * _CH, _CH)], sbuf,
                            isem).wait()
      pltpu.make_async_copy(dst_hbm.at[sid, pl.ds(c * _CH, _CH)], dbuf,
                            isem).wait()

    load_idx(0, sbuf0, dbuf0, isem0)

    @pl.when(sid < _NS - 1)
    def _():
      pltpu.sync_copy(
          table_hbm.at[pl.ds(sid * _RPT, _RPT), pl.ds(col0, _WSL)],
          table_sh.at[pl.ds(sid * _RPT, _RPT)])

    @pl.when(sid == _NS - 1)
    def _():
      last = _N - (_NS - 1) * _RPT
      pltpu.sync_copy(
          table_hbm.at[pl.ds((_NS - 1) * _RPT, last), pl.ds(col0, _WSL)],
          table_sh.at[pl.ds((_NS - 1) * _RPT, last)])
    _fill(rows0_v, _K, _WSL, 0, jnp.bfloat16, 32)
    _zero_slice(acc_sh, rows0_v, _K, sid)
    _fill(ones_v, _K, _DW, 0.0, jnp.float32, 16)
    _zero_slice(deg_sh, ones_v, _K, sid)
    _fill(ones_v, _K, _DW, 1.0, jnp.float32, 16)
    plsc.subcore_barrier()

    def scat(rows_v, dref, parity):
      pltpu.sync_copy(rows_v, acc_sh.at[dref], add=True)

      @pl.when(cid == parity)
      def _():
        pltpu.sync_copy(ones_v, deg_sh.at[dref], add=True)

    def chunk(c, sbuf, dbuf):
      pltpu.async_copy(table_sh.at[sbuf.at[0]], rows0_v, rsem0)

      def pair(i, carry):
        pltpu.async_copy(table_sh.at[sbuf.at[2 * i + 1]], rows1_v, rsem1)
        pltpu.make_async_copy(table_sh.at[sbuf.at[2 * i]], rows0_v,
                              rsem0).wait()
        scat(rows0_v, dbuf.at[2 * i], 0)

        @pl.when(2 * i + 2 < _CH)
        def _():
          pltpu.async_copy(table_sh.at[sbuf.at[2 * i + 2]], rows0_v, rsem0)

        pltpu.make_async_copy(table_sh.at[sbuf.at[2 * i + 1]], rows1_v,
                              rsem1).wait()
        scat(rows1_v, dbuf.at[2 * i + 1], 1)
        return carry

      lax.fori_loop(0, _CH // 2, pair, 0)

    def outer(m, carry):
      c0 = 2 * m
      c1 = 2 * m + 1
      wait_idx(c0, sbuf0, dbuf0, isem0)
      load_idx(c1, sbuf1, dbuf1, isem1)
      chunk(c0, sbuf0, dbuf0)
      wait_idx(c1, sbuf1, dbuf1, isem1)

      @pl.when(c1 + 1 < _NCH)
      def _():
        load_idx(c1 + 1, sbuf0, dbuf0, isem0)

      chunk(c1, sbuf1, dbuf1)
      return carry

    lax.fori_loop(0, _NCH // 2, outer, 0)

    plsc.subcore_barrier()
    rsl = pl.ds(sid * _RPT, _RPT)
    pltpu.sync_copy(acc_sh.at[rsl], out_hbm.at[rsl, pl.ds(col0, _WSL)])
    pltpu.sync_copy(deg_sh.at[rsl], deg_hbm.at[cid, rsl])

  return seg


def _make_seg1():
  @functools.partial(
      pl.kernel,
      mesh=_get_mesh(),
      compiler_params=pltpu.CompilerParams(use_tc_tiling_on_sc=False),
      out_type=jax.ShapeDtypeStruct((_NC, _NACC, _W1), jnp.bfloat16),
      scratch_types=[
          pltpu.VMEM((_NBLK1, _K), jnp.int32),
          pltpu.VMEM((_NBLK1, _K), jnp.int32),
          pltpu.VMEM((_K, _W1), jnp.bfloat16),
          pltpu.VMEM((_K, _W1), jnp.bfloat16),
          pltpu.VMEM_SHARED((_NACC, _W1), jnp.bfloat16),
          pltpu.VMEM_SHARED((_NACC, _W1), jnp.bfloat16),
          pltpu.SemaphoreType.DMA,
          pltpu.SemaphoreType.DMA,
      ],
  )
  def seg(table_hbm, src_hbm, dst_hbm, out_hbm,
          src_v, dst_v, rows0_v, rows1_v, table_sh, acc_sh, sem0, sem1):
    cid = lax.axis_index("c")
    sid = lax.axis_index("s")

    pltpu.sync_copy(src_hbm.at[cid, sid], src_v)
    pltpu.sync_copy(dst_hbm.at[cid, sid], dst_v)
    pltpu.sync_copy(table_hbm.at[pl.ds(sid * _RPT, _RPT)],
                    table_sh.at[pl.ds(sid * _RPT, _RPT)])
    z32 = jnp.zeros((32,), jnp.bfloat16)

    def zrow(i, carry):
      rows0_v[i, pl.ds(0, 32)] = z32
      rows0_v[i, pl.ds(_W1 - 32, 32)] = z32
      return carry

    lax.fori_loop(0, _K, zrow, 0)
    _zero_slice(acc_sh, rows0_v, _K, sid)
    plsc.subcore_barrier()

    pltpu.async_copy(table_sh.at[src_v.at[0]], rows0_v, sem0)

    def pair(i, carry):
      j = 2 * i
      pltpu.async_copy(table_sh.at[src_v.at[j + 1]], rows1_v, sem1)
      pltpu.make_async_copy(table_sh.at[src_v.at[j]], rows0_v, sem0).wait()
      pltpu.sync_copy(rows0_v, acc_sh.at[dst_v.at[j]], add=True)

      @pl.when(j + 2 < _NBLK1)
      def _():
        pltpu.async_copy(table_sh.at[src_v.at[j + 2]], rows0_v, sem0)

      pltpu.make_async_copy(table_sh.at[src_v.at[j + 1]], rows1_v,
                            sem1).wait()
      pltpu.sync_copy(rows1_v, acc_sh.at[dst_v.at[j + 1]], add=True)
      return carry

    lax.fori_loop(0, _NBLK1 // 2, pair, 0)

    plsc.subcore_barrier()
    pltpu.sync_copy(acc_sh.at[pl.ds(sid * _RPT, _RPT)],
                    out_hbm.at[cid, pl.ds(sid * _RPT, _RPT)])

  return seg


_sc_cache = {}


def _sc(kind):
  if kind not in _sc_cache:
    _sc_cache[kind] = _make_seg0() if kind == "seg0" else _make_seg1()
  return _sc_cache[kind]


_R = 1000


def _layerB_body(x_ref, p0_ref, deg_ref, ws0_ref, wn0_ref, b0_ref, ws1_ref,
                 wn1_ref, b1_ref, t1_ref, hse_ref):
  agg = p0_ref[...].astype(jnp.float32)
  deg = deg_ref[0, :, 0:1] + deg_ref[1, :, 0:1]
  inv = 1.0 / jnp.maximum(deg, 1.0)
  hn = jnp.dot(agg * inv, wn0_ref[...], preferred_element_type=jnp.float32)
  hself = jnp.dot(x_ref[...], ws0_ref[...].astype(jnp.bfloat16),
                  preferred_element_type=jnp.float32)
  h = hself + hn + b0_ref[...][None, :]
  h = jnp.maximum(h, 0.0)
  t1 = jnp.dot(h, wn1_ref[...], preferred_element_type=jnp.float32)
  hs = jnp.dot(h, ws1_ref[...], preferred_element_type=jnp.float32)
  hs = hs + b1_ref[...][None, :]
  t1_ref[...] = jnp.concatenate(
      [t1, jnp.zeros((_R, _W1 - _C), jnp.float32)], axis=1
  ).astype(jnp.bfloat16)
  zpad = jnp.zeros((_R, 7), jnp.float32)
  hse_ref[...] = jnp.concatenate([hs, inv, zpad], axis=1).astype(jnp.bfloat16)


def _layerC_body(p1_ref, hse_ref, o_ref):
  s = p1_ref[0].astype(jnp.float32) + p1_ref[1].astype(jnp.float32)
  hse = hse_ref[...].astype(jnp.float32)
  inv = hse[:, _C:_C + 1]
  o_ref[...] = hse[:, :_C] + s[:, :_C] * inv


def kernel(features, edge_index, W_self0, W_neigh0, b0, W_self1, W_neigh1, b1):
  src = edge_index[0]
  dst = edge_index[1]
  pad = _EPAD - _E
  src_p = jnp.concatenate([src, jnp.full((pad,), _N, jnp.int32)])
  dst_p = jnp.concatenate([dst, jnp.full((pad,), _N, jnp.int32)])
  src_cs = src_p.reshape(_NS, _NBLK0, _K)
  dst_cs = dst_p.reshape(_NS, _NBLK0, _K)
  src_r = src_p.reshape(_NC, _NS, _NBLK1, _K)
  dst_r = dst_p.reshape(_NC, _NS, _NBLK1, _K)

  nb = _N // _R

  xb = features.astype(jnp.bfloat16)
  p0, deg = _sc("seg0")(xb, src_cs, dst_cs)

  t1, hse = pl.pallas_call(
      _layerB_body,
      grid=(nb,),
      in_specs=[
          pl.BlockSpec((_R, _D), lambda i: (i, 0)),
          pl.BlockSpec((_R, _W0), lambda i: (i, 0)),
          pl.BlockSpec((_NC, _R, _DW), lambda i: (0, i, 0)),
          pl.BlockSpec((_D, _H), lambda i: (0, 0)),
          pl.BlockSpec((_D, _H), lambda i: (0, 0)),
          pl.BlockSpec((_H,), lambda i: (0,)),
          pl.BlockSpec((_H, _C), lambda i: (0, 0)),
          pl.BlockSpec((_H, _C), lambda i: (0, 0)),
          pl.BlockSpec((_C,), lambda i: (0,)),
      ],
      out_specs=[
          pl.BlockSpec((_R, _W1), lambda i: (i, 0)),
          pl.BlockSpec((_R, _C + 8), lambda i: (i, 0)),
      ],
      out_shape=[
          jax.ShapeDtypeStruct((_NACC, _W1), jnp.bfloat16),
          jax.ShapeDtypeStruct((_N, _C + 8), jnp.bfloat16),
      ],
  )(xb, p0, deg, W_self0, W_neigh0, b0, W_self1, W_neigh1, b1)

  p1 = _sc("seg1")(t1, src_r, dst_r)

  out = pl.pallas_call(
      _layerC_body,
      grid=(nb,),
      in_specs=[
          pl.BlockSpec((_NC, _R, _W1), lambda i: (0, i, 0)),
          pl.BlockSpec((_R, _C + 8), lambda i: (i, 0)),
      ],
      out_specs=pl.BlockSpec((_R, _C), lambda i: (i, 0)),
      out_shape=jax.ShapeDtypeStruct((_N, _C), jnp.float32),
  )(p1, hse)

  return out
